# baseline (speedup 1.0000x reference)
import functools

import jax
import jax.numpy as jnp
from jax import lax
from jax.experimental import pallas as pl
from jax.experimental.pallas import tpu as pltpu
from jax.experimental.pallas import tpu_sc as plsc

USER_N = 25000
ITEM_N = 25000
NODES = USER_N + ITEM_N
EMB = 64
HALF = EMB // 2
LAYERS = 3
EDGES = 800000

NC = 2
NS = 16
LANES = 16
CHUNK = 128
NSLOT = 2
NP = 50048
ROWS_PER_TILE = NP // NS
PIECE = 136
NPIECE = ROWS_PER_TILE // PIECE
TAIL = NODES - (NP - ROWS_PER_TILE) - (NPIECE - 1) * PIECE
_NCH = -(-EDGES // (NS * CHUNK))
NCHUNK = -(-_NCH // NSLOT) * NSLOT
NCHUNK_ALLOC = NCHUNK + NSLOT
SLAB = NC * NP


def _build_kernel():
    mesh = plsc.VectorSubcoreMesh(core_axis_name="c", subcore_axis_name="s")

    @functools.partial(
        pl.kernel,
        out_type=(
            jax.ShapeDtypeStruct(((LAYERS + 1) * SLAB, HALF), jnp.float32),
            jax.ShapeDtypeStruct((LAYERS + 1, NODES, EMB), jnp.float32),
            jax.ShapeDtypeStruct((NODES, EMB), jnp.float32),
        ),
        mesh=mesh,
        compiler_params=pltpu.CompilerParams(use_tc_tiling_on_sc=False),
        scratch_types=[
            [pltpu.VMEM((2, CHUNK), jnp.int32) for _ in range(NSLOT)],
            [pltpu.VMEM((1, CHUNK), jnp.float32) for _ in range(NSLOT)],
            [pltpu.VMEM((CHUNK, HALF), jnp.float32) for _ in range(NSLOT)],
            pltpu.VMEM((PIECE, HALF), jnp.float32),
            pltpu.VMEM((PIECE, HALF), jnp.float32),
            pltpu.VMEM_SHARED((NP, HALF), jnp.float32),
            [pltpu.SemaphoreType.DMA for _ in range(NSLOT)],
            [pltpu.SemaphoreType.DMA for _ in range(NSLOT)],
        ],
    )
    def lgcn_kernel(table0, blob, vals, t_out, all_out, mean_out,
                    ebufs, vbufs, rowss,
                    pbuf, p1, acc,
                    esems, gsems):
        c = lax.axis_index("c")
        s = lax.axis_index("s")
        half_off = c * NP
        r0 = s * ROWS_PER_TILE

        zv = jnp.zeros((LANES,), jnp.float32)
        for r in range(PIECE):
            for h in range(HALF // LANES):
                p1[r, pl.ds(h * LANES, LANES)] = zv

        def e_start(k, j):
            pltpu.async_copy(blob.at[s, j], ebufs[k], esems[k])
            pltpu.async_copy(vals.at[s, j], vbufs[k], esems[k])

        def e_wait(k, j):
            pltpu.make_async_copy(blob.at[s, j], ebufs[k], esems[k]).wait()
            pltpu.make_async_copy(vals.at[s, j], vbufs[k], esems[k]).wait()

        def adjust(k, gather_base):
            eb = ebufs[k]
            for q in range(CHUNK // LANES):
                sl = pl.ds(q * LANES, LANES)
                eb[0, sl] = eb[0, sl] + gather_base

        def g_start(k):
            pltpu.async_copy(t_out.at[ebufs[k].at[0]], rowss[k], gsems[k])

        def g_wait(k):
            pltpu.make_async_copy(t_out.at[ebufs[k].at[0]], rowss[k],
                                  gsems[k]).wait()

        def mul_rows(k):
            rows = rowss[k]
            vb = vbufs[k]
            for g in range(CHUNK // LANES):
                sl16 = pl.ds(g * LANES, LANES)
                vv = vb[0, sl16]
                for e in range(LANES):
                    b = jnp.take_along_axis(
                        vv, jnp.full((LANES,), e, jnp.int32), axis=0)
                    r = g * LANES + e
                    for h in range(HALF // LANES):
                        sl = pl.ds(h * LANES, LANES)
                        rows[r, sl] = rows[r, sl] * b

        def scatter(k):
            pltpu.sync_copy(rowss[k], acc.at[ebufs[k].at[1]], add=True)

        def zero_acc():
            def zbody(p, _):
                pltpu.sync_copy(p1, acc.at[pl.ds(r0 + p * PIECE, PIECE)])
                return _
            lax.fori_loop(0, NPIECE, zbody, 0)

        def write_final(dst_is_mean, li, roff, src):
            full = roff <= NODES - PIECE
            for ci in range(NC):
                sel = jnp.logical_and(full, c == ci)

                @pl.when(sel)
                def _():
                    cs = pl.ds(ci * HALF, HALF)
                    if dst_is_mean:
                        dst = mean_out.at[pl.ds(roff, PIECE), cs]
                    else:
                        dst = all_out.at[li, pl.ds(roff, PIECE), cs]
                    pltpu.sync_copy(src, dst)

                selt = jnp.logical_and(jnp.logical_not(full), c == ci)

                @pl.when(selt)
                def _():
                    cs = pl.ds(ci * HALF, HALF)
                    if dst_is_mean:
                        dst = mean_out.at[pl.ds(roff, TAIL), cs]
                    else:
                        dst = all_out.at[li, pl.ds(roff, TAIL), cs]
                    pltpu.sync_copy(src.at[pl.ds(0, TAIL)], dst)

        def writeback(li, out_base):
            def wbody(p, _):
                off = r0 + p * PIECE
                pltpu.sync_copy(acc.at[pl.ds(off, PIECE)], pbuf)
                pltpu.sync_copy(
                    pbuf, t_out.at[pl.ds(out_base + half_off + off, PIECE)])
                write_final(False, li, off, pbuf)
                return _
            lax.fori_loop(0, NPIECE, wbody, 0)

        def cbody(p, _):
            off = r0 + p * PIECE
            pltpu.sync_copy(table0.at[pl.ds(half_off + off, PIECE)], pbuf)
            pltpu.sync_copy(pbuf, t_out.at[pl.ds(half_off + off, PIECE)])
            write_final(False, 0, off, pbuf)
            return _
        lax.fori_loop(0, NPIECE, cbody, 0)
        plsc.subcore_barrier()

        def layer(i, _):
            gather_base = half_off + i * SLAB
            out_base = (i + 1) * SLAB
            zero_acc()
            plsc.subcore_barrier()

            e_start(0, 0)
            e_wait(0, 0)
            adjust(0, gather_base)
            g_start(0)
            e_start(1, 1)
            g_wait(0)

            def ebody(jj, _):
                j0 = NSLOT * jj
                for k in range(NSLOT):
                    j = j0 + k
                    o = (k + 1) % NSLOT
                    e_wait(o, j + 1)
                    adjust(o, gather_base)
                    g_start(o)
                    mul_rows(k)
                    g_wait(o)
                    scatter(k)
                    e_start(k, j + 2)
                return _
            lax.fori_loop(0, NCHUNK // NSLOT, ebody, 0)
            e_wait(1, NCHUNK + 1)

            plsc.subcore_barrier()
            writeback(i + 1, out_base)
            plsc.subcore_barrier()
            return _
        lax.fori_loop(0, LAYERS, layer, 0)

        def mbody(p, _):
            off = r0 + p * PIECE
            hb = half_off + off
            pltpu.sync_copy(t_out.at[pl.ds(hb, PIECE)], pbuf)
            for li in (1, 2, 3):
                pltpu.sync_copy(t_out.at[pl.ds(li * SLAB + hb, PIECE)], p1)
                scale = 0.25 if li == 3 else 1.0

                def rbody(r, _, scale=scale):
                    for h in range(HALF // LANES):
                        sl = pl.ds(h * LANES, LANES)
                        pbuf[r, sl] = (pbuf[r, sl] + p1[r, sl]) * scale
                    return _
                lax.fori_loop(0, PIECE, rbody, 0)
            write_final(True, 0, off, pbuf)
            return _
        lax.fori_loop(0, NPIECE, mbody, 0)

    return lgcn_kernel


_LGCN = _build_kernel()


def kernel(user_emb, item_emb, adj_values, adj_indices):
    emb0 = jnp.concatenate([user_emb, item_emb], axis=0)
    rpad = NP - NODES
    table0 = jnp.concatenate(
        [jnp.pad(emb0[:, :HALF], ((0, rpad), (0, 0))),
         jnp.pad(emb0[:, HALF:], ((0, rpad), (0, 0)))], axis=0)

    row = adj_indices[0]
    col = adj_indices[1]
    pad = NS * NCHUNK * CHUNK - EDGES
    colp = jnp.pad(col, (0, pad))
    rowp = jnp.pad(row, (0, pad))
    valp = jnp.pad(adj_values, (0, pad))
    blob = jnp.stack([colp, rowp], axis=0)
    blob = blob.reshape(2, NS, NCHUNK, CHUNK).transpose(1, 2, 0, 3)
    blob = jnp.pad(blob, ((0, 0), (0, NCHUNK_ALLOC - NCHUNK), (0, 0), (0, 0)))
    vals = valp.reshape(NS, NCHUNK, 1, CHUNK)
    vals = jnp.pad(vals, ((0, 0), (0, NCHUNK_ALLOC - NCHUNK), (0, 0), (0, 0)))

    _, all_emb, final = _LGCN(table0, blob, vals)
    return (final[:USER_N], final[USER_N:], all_emb)


# speedup vs baseline: 1.3470x; 1.3470x over previous
import functools

import jax
import jax.numpy as jnp
from jax import lax
from jax.experimental import pallas as pl
from jax.experimental.pallas import tpu as pltpu
from jax.experimental.pallas import tpu_sc as plsc

USER_N = 25000
ITEM_N = 25000
NODES = USER_N + ITEM_N
EMB = 64
HALF = EMB // 2
LAYERS = 3
EDGES = 800000

NC = 2
NS = 16
LANES = 16
CHUNK = 128
NSLOT = 2
NP = 50048
ROWS_PER_TILE = NP // NS
PIECE = 136
NPIECE = ROWS_PER_TILE // PIECE
TAIL = NODES - (NP - ROWS_PER_TILE) - (NPIECE - 1) * PIECE
_NCH = -(-EDGES // (NS * CHUNK))
NCHUNK = -(-_NCH // NSLOT) * NSLOT
NCHUNK_ALLOC = NCHUNK + NSLOT
SLAB = NC * NP


def _build_kernel():
    mesh = plsc.VectorSubcoreMesh(core_axis_name="c", subcore_axis_name="s")

    @functools.partial(
        pl.kernel,
        out_type=(
            jax.ShapeDtypeStruct(((LAYERS + 1) * SLAB, HALF), jnp.float32),
            jax.ShapeDtypeStruct((LAYERS + 1, NODES, EMB), jnp.float32),
            jax.ShapeDtypeStruct((NODES, EMB), jnp.float32),
        ),
        mesh=mesh,
        compiler_params=pltpu.CompilerParams(use_tc_tiling_on_sc=False),
        scratch_types=[
            [pltpu.VMEM((2, CHUNK), jnp.int32) for _ in range(NSLOT)],
            [pltpu.VMEM((1, CHUNK), jnp.float32) for _ in range(NSLOT)],
            [pltpu.VMEM((CHUNK, HALF), jnp.float32) for _ in range(NSLOT)],
            pltpu.VMEM((PIECE, HALF), jnp.float32),
            pltpu.VMEM((PIECE, HALF), jnp.float32),
            pltpu.VMEM_SHARED((NP, HALF), jnp.float32),
            [pltpu.SemaphoreType.DMA for _ in range(NSLOT)],
            [pltpu.SemaphoreType.DMA for _ in range(NSLOT)],
        ],
    )
    def lgcn_kernel(table0, blob, vals, t_out, all_out, mean_out,
                    ebufs, vbufs, rowss,
                    pbuf, p1, acc,
                    esems, gsems):
        c = lax.axis_index("c")
        s = lax.axis_index("s")
        half_off = c * NP
        r0 = s * ROWS_PER_TILE

        zv = jnp.zeros((LANES,), jnp.float32)
        for r in range(PIECE):
            for h in range(HALF // LANES):
                p1[r, pl.ds(h * LANES, LANES)] = zv

        def e_start(k, j):
            pltpu.async_copy(blob.at[s, j], ebufs[k], esems[k])
            pltpu.async_copy(vals.at[s, j], vbufs[k], esems[k])

        def e_wait(k, j):
            pltpu.make_async_copy(blob.at[s, j], ebufs[k], esems[k]).wait()
            pltpu.make_async_copy(vals.at[s, j], vbufs[k], esems[k]).wait()

        def adjust(k, gather_base):
            eb = ebufs[k]
            for q in range(CHUNK // LANES):
                sl = pl.ds(q * LANES, LANES)
                eb[0, sl] = eb[0, sl] + gather_base

        def g_start(k):
            pltpu.async_copy(t_out.at[ebufs[k].at[0]], rowss[k], gsems[k])

        def g_wait(k):
            pltpu.make_async_copy(t_out.at[ebufs[k].at[0]], rowss[k],
                                  gsems[k]).wait()

        def mul_rows(k):
            rows = rowss[k]
            vb = vbufs[k]
            for g in range(CHUNK // LANES):
                sl16 = pl.ds(g * LANES, LANES)
                vv = vb[0, sl16]
                for e in range(LANES):
                    b = jnp.take_along_axis(
                        vv, jnp.full((LANES,), e, jnp.int32), axis=0)
                    r = g * LANES + e
                    for h in range(HALF // LANES):
                        sl = pl.ds(h * LANES, LANES)
                        rows[r, sl] = rows[r, sl] * b

        def scatter(k):
            pltpu.sync_copy(rowss[k], acc.at[ebufs[k].at[1]], add=True)

        def zero_acc():
            def zbody(p, _):
                pltpu.sync_copy(p1, acc.at[pl.ds(r0 + p * PIECE, PIECE)])
                return _
            lax.fori_loop(0, NPIECE, zbody, 0)

        def write_final(dst_is_mean, li, roff, src):
            full = roff <= NODES - PIECE
            for ci in range(NC):
                sel = jnp.logical_and(full, c == ci)

                @pl.when(sel)
                def _():
                    cs = pl.ds(ci * HALF, HALF)
                    if dst_is_mean:
                        dst = mean_out.at[pl.ds(roff, PIECE), cs]
                    else:
                        dst = all_out.at[li, pl.ds(roff, PIECE), cs]
                    pltpu.sync_copy(src, dst)

                selt = jnp.logical_and(jnp.logical_not(full), c == ci)

                @pl.when(selt)
                def _():
                    cs = pl.ds(ci * HALF, HALF)
                    if dst_is_mean:
                        dst = mean_out.at[pl.ds(roff, TAIL), cs]
                    else:
                        dst = all_out.at[li, pl.ds(roff, TAIL), cs]
                    pltpu.sync_copy(src.at[pl.ds(0, TAIL)], dst)

        def writeback(li, out_base):
            def wbody(p, _):
                off = r0 + p * PIECE
                pltpu.sync_copy(acc.at[pl.ds(off, PIECE)], pbuf)
                pltpu.sync_copy(
                    pbuf, t_out.at[pl.ds(out_base + half_off + off, PIECE)])
                write_final(False, li, off, pbuf)
                return _
            lax.fori_loop(0, NPIECE, wbody, 0)

        def cbody(p, _):
            off = r0 + p * PIECE
            pltpu.sync_copy(table0.at[pl.ds(half_off + off, PIECE)], pbuf)
            pltpu.sync_copy(pbuf, t_out.at[pl.ds(half_off + off, PIECE)])
            write_final(False, 0, off, pbuf)
            return _
        lax.fori_loop(0, NPIECE, cbody, 0)
        plsc.subcore_barrier()

        def layer(i, _):
            gather_base = half_off + i * SLAB
            out_base = (i + 1) * SLAB
            zero_acc()
            plsc.subcore_barrier()

            e_start(0, 0)
            e_wait(0, 0)
            adjust(0, gather_base)
            g_start(0)
            e_start(1, 1)

            def ebody(jj, _):
                j0 = NSLOT * jj
                for k in range(NSLOT):
                    j = j0 + k
                    o = (k + 1) % NSLOT
                    g_wait(k)
                    mul_rows(k)
                    e_wait(o, j + 1)
                    adjust(o, gather_base)
                    g_start(o)
                    scatter(k)
                    e_start(k, j + 2)
                return _
            lax.fori_loop(0, NCHUNK // NSLOT, ebody, 0)
            g_wait(0)
            e_wait(1, NCHUNK + 1)

            plsc.subcore_barrier()
            writeback(i + 1, out_base)
            plsc.subcore_barrier()
            return _
        lax.fori_loop(0, LAYERS, layer, 0)

        def mbody(p, _):
            off = r0 + p * PIECE
            hb = half_off + off
            pltpu.sync_copy(t_out.at[pl.ds(hb, PIECE)], pbuf)
            for li in (1, 2, 3):
                pltpu.sync_copy(t_out.at[pl.ds(li * SLAB + hb, PIECE)], p1)
                scale = 0.25 if li == 3 else 1.0

                def rbody(r, _, scale=scale):
                    for h in range(HALF // LANES):
                        sl = pl.ds(h * LANES, LANES)
                        pbuf[r, sl] = (pbuf[r, sl] + p1[r, sl]) * scale
                    return _
                lax.fori_loop(0, PIECE, rbody, 0)
            write_final(True, 0, off, pbuf)
            return _
        lax.fori_loop(0, NPIECE, mbody, 0)

    return lgcn_kernel


_LGCN = _build_kernel()


def kernel(user_emb, item_emb, adj_values, adj_indices):
    emb0 = jnp.concatenate([user_emb, item_emb], axis=0)
    rpad = NP - NODES
    table0 = jnp.concatenate(
        [jnp.pad(emb0[:, :HALF], ((0, rpad), (0, 0))),
         jnp.pad(emb0[:, HALF:], ((0, rpad), (0, 0)))], axis=0)

    row = adj_indices[0]
    col = adj_indices[1]
    pad = NS * NCHUNK * CHUNK - EDGES
    colp = jnp.pad(col, (0, pad))
    rowp = jnp.pad(row, (0, pad))
    valp = jnp.pad(adj_values, (0, pad))
    blob = jnp.stack([colp, rowp], axis=0)
    blob = blob.reshape(2, NS, NCHUNK, CHUNK).transpose(1, 2, 0, 3)
    blob = jnp.pad(blob, ((0, 0), (0, NCHUNK_ALLOC - NCHUNK), (0, 0), (0, 0)))
    vals = valp.reshape(NS, NCHUNK, 1, CHUNK)
    vals = jnp.pad(vals, ((0, 0), (0, NCHUNK_ALLOC - NCHUNK), (0, 0), (0, 0)))

    _, all_emb, final = _LGCN(table0, blob, vals)
    return (final[:USER_N], final[USER_N:], all_emb)


# speedup vs baseline: 1.3999x; 1.0393x over previous
import functools

import jax
import jax.numpy as jnp
from jax import lax
from jax.experimental import pallas as pl
from jax.experimental.pallas import tpu as pltpu
from jax.experimental.pallas import tpu_sc as plsc

USER_N = 25000
ITEM_N = 25000
NODES = USER_N + ITEM_N
EMB = 64
HALF = EMB // 2
LAYERS = 3
EDGES = 800000

NC = 2
NS = 16
LANES = 16
CHUNK = 128
NSLOT = 2
NP = 50048
ROWS_PER_TILE = NP // NS
PIECE = 136
NPIECE = ROWS_PER_TILE // PIECE
TAIL = NODES - (NP - ROWS_PER_TILE) - (NPIECE - 1) * PIECE
_NCH = -(-EDGES // (NS * CHUNK))
NCHUNK = -(-_NCH // NSLOT) * NSLOT
NCHUNK_ALLOC = NCHUNK + NSLOT
SLAB = NC * NP


def _build_kernel():
    mesh = plsc.VectorSubcoreMesh(core_axis_name="c", subcore_axis_name="s")

    @functools.partial(
        pl.kernel,
        out_type=(
            jax.ShapeDtypeStruct(((LAYERS + 1) * SLAB, HALF), jnp.float32),
            jax.ShapeDtypeStruct((LAYERS + 1, NODES, EMB), jnp.float32),
            jax.ShapeDtypeStruct((NODES, EMB), jnp.float32),
        ),
        mesh=mesh,
        compiler_params=pltpu.CompilerParams(use_tc_tiling_on_sc=False),
        scratch_types=[
            [pltpu.VMEM((2, CHUNK), jnp.int32) for _ in range(NSLOT)],
            [pltpu.VMEM((1, CHUNK), jnp.float32) for _ in range(NSLOT)],
            [pltpu.VMEM((CHUNK, HALF), jnp.float32) for _ in range(NSLOT)],
            pltpu.VMEM((PIECE, HALF), jnp.float32),
            pltpu.VMEM((PIECE, HALF), jnp.float32),
            pltpu.VMEM_SHARED((NP, HALF), jnp.float32),
            [pltpu.SemaphoreType.DMA for _ in range(NSLOT)],
            [pltpu.SemaphoreType.DMA for _ in range(NSLOT)],
        ],
    )
    def lgcn_kernel(table0, blob, vals, t_out, all_out, mean_out,
                    ebufs, vbufs, rowss,
                    pbuf, p1, acc,
                    esems, gsems):
        c = lax.axis_index("c")
        s = lax.axis_index("s")
        half_off = c * NP
        r0 = s * ROWS_PER_TILE

        zv = jnp.zeros((LANES,), jnp.float32)
        for r in range(PIECE):
            for h in range(HALF // LANES):
                p1[r, pl.ds(h * LANES, LANES)] = zv

        def e_start(k, j):
            pltpu.async_copy(blob.at[s, j], ebufs[k], esems[k])
            pltpu.async_copy(vals.at[s, j], vbufs[k], esems[k])

        def e_wait(k, j):
            pltpu.make_async_copy(blob.at[s, j], ebufs[k], esems[k]).wait()
            pltpu.make_async_copy(vals.at[s, j], vbufs[k], esems[k]).wait()

        def adjust(k, gather_base):
            eb = ebufs[k]
            for q in range(CHUNK // LANES):
                sl = pl.ds(q * LANES, LANES)
                eb[0, sl] = eb[0, sl] + gather_base

        def g_start(k):
            pltpu.async_copy(t_out.at[ebufs[k].at[0]], rowss[k], gsems[k])

        def g_wait(k):
            pltpu.make_async_copy(t_out.at[ebufs[k].at[0]], rowss[k],
                                  gsems[k]).wait()

        def mul_rows(k):
            rows = rowss[k]
            vb = vbufs[k]
            for g in range(CHUNK // LANES):
                sl16 = pl.ds(g * LANES, LANES)
                vv = vb[0, sl16]
                for e in range(LANES):
                    b = jnp.take_along_axis(
                        vv, jnp.full((LANES,), e, jnp.int32), axis=0)
                    r = g * LANES + e
                    for h in range(HALF // LANES):
                        sl = pl.ds(h * LANES, LANES)
                        rows[r, sl] = rows[r, sl] * b

        def scatter(k):
            pltpu.sync_copy(rowss[k], acc.at[ebufs[k].at[1]], add=True)

        def zero_acc():
            def zbody(p, _):
                pltpu.sync_copy(p1, acc.at[pl.ds(r0 + p * PIECE, PIECE)])
                return _
            lax.fori_loop(0, NPIECE, zbody, 0)

        def write_final(dst_is_mean, li, roff, src):
            full = roff <= NODES - PIECE
            for ci in range(NC):
                sel = jnp.logical_and(full, c == ci)

                @pl.when(sel)
                def _():
                    cs = pl.ds(ci * HALF, HALF)
                    if dst_is_mean:
                        dst = mean_out.at[pl.ds(roff, PIECE), cs]
                    else:
                        dst = all_out.at[li, pl.ds(roff, PIECE), cs]
                    pltpu.sync_copy(src, dst)

                selt = jnp.logical_and(jnp.logical_not(full), c == ci)

                @pl.when(selt)
                def _():
                    cs = pl.ds(ci * HALF, HALF)
                    if dst_is_mean:
                        dst = mean_out.at[pl.ds(roff, TAIL), cs]
                    else:
                        dst = all_out.at[li, pl.ds(roff, TAIL), cs]
                    pltpu.sync_copy(src.at[pl.ds(0, TAIL)], dst)

        def writeback(li, out_base):
            def wbody(p, _):
                off = r0 + p * PIECE
                pltpu.sync_copy(acc.at[pl.ds(off, PIECE)], pbuf)
                pltpu.sync_copy(
                    pbuf, t_out.at[pl.ds(out_base + half_off + off, PIECE)])
                write_final(False, li, off, pbuf)
                return _
            lax.fori_loop(0, NPIECE, wbody, 0)

        def cbody(p, _):
            off = r0 + p * PIECE
            pltpu.sync_copy(table0.at[pl.ds(half_off + off, PIECE)], pbuf)
            pltpu.sync_copy(pbuf, t_out.at[pl.ds(half_off + off, PIECE)])
            write_final(False, 0, off, pbuf)
            return _
        lax.fori_loop(0, NPIECE, cbody, 0)
        plsc.subcore_barrier()

        def layer(i, _):
            gather_base = half_off + i * SLAB
            out_base = (i + 1) * SLAB
            zero_acc()
            plsc.subcore_barrier()

            e_start(0, 0)
            e_wait(0, 0)
            adjust(0, gather_base)
            g_start(0)
            e_start(1, 1)

            def ebody(jj, _):
                j0 = NSLOT * jj
                for k in range(NSLOT):
                    j = j0 + k
                    o = (k + 1) % NSLOT
                    g_wait(k)
                    e_wait(o, j + 1)
                    adjust(o, gather_base)
                    g_start(o)
                    mul_rows(k)
                    scatter(k)
                    e_start(k, j + 2)
                return _
            lax.fori_loop(0, NCHUNK // NSLOT, ebody, 0)
            g_wait(0)
            e_wait(1, NCHUNK + 1)

            plsc.subcore_barrier()
            writeback(i + 1, out_base)
            plsc.subcore_barrier()
            return _
        lax.fori_loop(0, LAYERS, layer, 0)

        def mbody(p, _):
            off = r0 + p * PIECE
            hb = half_off + off
            pltpu.sync_copy(t_out.at[pl.ds(hb, PIECE)], pbuf)
            for li in (1, 2, 3):
                pltpu.sync_copy(t_out.at[pl.ds(li * SLAB + hb, PIECE)], p1)
                scale = 0.25 if li == 3 else 1.0

                def rbody(r, _, scale=scale):
                    for h in range(HALF // LANES):
                        sl = pl.ds(h * LANES, LANES)
                        pbuf[r, sl] = (pbuf[r, sl] + p1[r, sl]) * scale
                    return _
                lax.fori_loop(0, PIECE, rbody, 0)
            write_final(True, 0, off, pbuf)
            return _
        lax.fori_loop(0, NPIECE, mbody, 0)

    return lgcn_kernel


_LGCN = _build_kernel()


def kernel(user_emb, item_emb, adj_values, adj_indices):
    emb0 = jnp.concatenate([user_emb, item_emb], axis=0)
    rpad = NP - NODES
    table0 = jnp.concatenate(
        [jnp.pad(emb0[:, :HALF], ((0, rpad), (0, 0))),
         jnp.pad(emb0[:, HALF:], ((0, rpad), (0, 0)))], axis=0)

    row = adj_indices[0]
    col = adj_indices[1]
    pad = NS * NCHUNK * CHUNK - EDGES
    colp = jnp.pad(col, (0, pad))
    rowp = jnp.pad(row, (0, pad))
    valp = jnp.pad(adj_values, (0, pad))
    blob = jnp.stack([colp, rowp], axis=0)
    blob = blob.reshape(2, NS, NCHUNK, CHUNK).transpose(1, 2, 0, 3)
    blob = jnp.pad(blob, ((0, 0), (0, NCHUNK_ALLOC - NCHUNK), (0, 0), (0, 0)))
    vals = valp.reshape(NS, NCHUNK, 1, CHUNK)
    vals = jnp.pad(vals, ((0, 0), (0, NCHUNK_ALLOC - NCHUNK), (0, 0), (0, 0)))

    _, all_emb, final = _LGCN(table0, blob, vals)
    return (final[:USER_N], final[USER_N:], all_emb)


# speedup vs baseline: 1.4005x; 1.0004x over previous
import functools

import jax
import jax.numpy as jnp
from jax import lax
from jax.experimental import pallas as pl
from jax.experimental.pallas import tpu as pltpu
from jax.experimental.pallas import tpu_sc as plsc

USER_N = 25000
ITEM_N = 25000
NODES = USER_N + ITEM_N
EMB = 64
HALF = EMB // 2
LAYERS = 3
EDGES = 800000

NC = 2
NS = 16
LANES = 16
CHUNK = 128
NSLOT = 2
NP = 50048
ROWS_PER_TILE = NP // NS
PIECE = 136
NPIECE = ROWS_PER_TILE // PIECE
TAIL = NODES - (NP - ROWS_PER_TILE) - (NPIECE - 1) * PIECE
_NCH = -(-EDGES // (NS * CHUNK))
NCHUNK = -(-_NCH // NSLOT) * NSLOT
NCHUNK_ALLOC = NCHUNK + NSLOT
SLAB = NC * NP


def _build_kernel():
    mesh = plsc.VectorSubcoreMesh(core_axis_name="c", subcore_axis_name="s")

    @functools.partial(
        pl.kernel,
        out_type=(
            jax.ShapeDtypeStruct(((LAYERS + 1) * SLAB, HALF), jnp.float32),
            jax.ShapeDtypeStruct((LAYERS + 1, NODES, EMB), jnp.float32),
            jax.ShapeDtypeStruct((NODES, EMB), jnp.float32),
        ),
        mesh=mesh,
        compiler_params=pltpu.CompilerParams(use_tc_tiling_on_sc=False),
        scratch_types=[
            [pltpu.VMEM((2, CHUNK), jnp.int32) for _ in range(NSLOT)],
            [pltpu.VMEM((1, CHUNK), jnp.float32) for _ in range(NSLOT)],
            [pltpu.VMEM((CHUNK, HALF), jnp.float32) for _ in range(NSLOT)],
            pltpu.VMEM((PIECE, HALF), jnp.float32),
            pltpu.VMEM((PIECE, HALF), jnp.float32),
            pltpu.VMEM_SHARED((NP, HALF), jnp.float32),
            [pltpu.SemaphoreType.DMA for _ in range(NSLOT)],
            [pltpu.SemaphoreType.DMA for _ in range(NSLOT)],
        ],
    )
    def lgcn_kernel(table0, blob, vals, t_out, all_out, mean_out,
                    ebufs, vbufs, rowss,
                    pbuf, p1, acc,
                    esems, gsems):
        c = lax.axis_index("c")
        s = lax.axis_index("s")
        half_off = c * NP
        r0 = s * ROWS_PER_TILE

        zv = jnp.zeros((LANES,), jnp.float32)
        for r in range(PIECE):
            for h in range(HALF // LANES):
                p1[r, pl.ds(h * LANES, LANES)] = zv

        def e_start(k, j):
            pltpu.async_copy(blob.at[s, j], ebufs[k], esems[k])
            pltpu.async_copy(vals.at[s, j], vbufs[k], esems[k])

        def e_wait(k, j):
            pltpu.make_async_copy(blob.at[s, j], ebufs[k], esems[k]).wait()
            pltpu.make_async_copy(vals.at[s, j], vbufs[k], esems[k]).wait()

        def adjust(k, gather_base):
            eb = ebufs[k]
            for q in range(CHUNK // LANES):
                sl = pl.ds(q * LANES, LANES)
                eb[0, sl] = eb[0, sl] + gather_base

        def g_start(k):
            pltpu.async_copy(t_out.at[ebufs[k].at[0]], rowss[k], gsems[k])

        def g_wait(k):
            pltpu.make_async_copy(t_out.at[ebufs[k].at[0]], rowss[k],
                                  gsems[k]).wait()

        def mul_rows(k):
            rows = rowss[k]
            vb = vbufs[k]
            for g in range(CHUNK // LANES):
                sl16 = pl.ds(g * LANES, LANES)
                vv = vb[0, sl16]
                for e in range(LANES):
                    b = jnp.take_along_axis(
                        vv, jnp.full((LANES,), e, jnp.int32), axis=0)
                    r = g * LANES + e
                    for h in range(HALF // LANES):
                        sl = pl.ds(h * LANES, LANES)
                        rows[r, sl] = rows[r, sl] * b

        def scatter(k):
            pltpu.sync_copy(rowss[k], acc.at[ebufs[k].at[1]], add=True)

        def zero_acc():
            def zbody(p, _):
                pltpu.async_copy(p1, acc.at[pl.ds(r0 + p * PIECE, PIECE)],
                                 esems[0])
                return _
            lax.fori_loop(0, NPIECE, zbody, 0)

            def zdrain(p, _):
                pltpu.make_async_copy(
                    p1, acc.at[pl.ds(r0 + p * PIECE, PIECE)],
                    esems[0]).wait()
                return _
            lax.fori_loop(0, NPIECE, zdrain, 0)

        def write_final(dst_is_mean, li, roff, src):
            full = roff <= NODES - PIECE
            for ci in range(NC):
                sel = jnp.logical_and(full, c == ci)

                @pl.when(sel)
                def _():
                    cs = pl.ds(ci * HALF, HALF)
                    if dst_is_mean:
                        dst = mean_out.at[pl.ds(roff, PIECE), cs]
                    else:
                        dst = all_out.at[li, pl.ds(roff, PIECE), cs]
                    pltpu.sync_copy(src, dst)

                selt = jnp.logical_and(jnp.logical_not(full), c == ci)

                @pl.when(selt)
                def _():
                    cs = pl.ds(ci * HALF, HALF)
                    if dst_is_mean:
                        dst = mean_out.at[pl.ds(roff, TAIL), cs]
                    else:
                        dst = all_out.at[li, pl.ds(roff, TAIL), cs]
                    pltpu.sync_copy(src.at[pl.ds(0, TAIL)], dst)

        def writeback(li, out_base):
            def wbody(p, _):
                off = r0 + p * PIECE
                pltpu.sync_copy(acc.at[pl.ds(off, PIECE)], pbuf)
                pltpu.sync_copy(
                    pbuf, t_out.at[pl.ds(out_base + half_off + off, PIECE)])
                write_final(False, li, off, pbuf)
                return _
            lax.fori_loop(0, NPIECE, wbody, 0)

        def cbody(p, _):
            off = r0 + p * PIECE
            pltpu.sync_copy(table0.at[pl.ds(half_off + off, PIECE)], pbuf)
            pltpu.sync_copy(pbuf, t_out.at[pl.ds(half_off + off, PIECE)])
            write_final(False, 0, off, pbuf)
            return _
        lax.fori_loop(0, NPIECE, cbody, 0)
        plsc.subcore_barrier()

        def layer(i, _):
            gather_base = half_off + i * SLAB
            out_base = (i + 1) * SLAB
            zero_acc()
            plsc.subcore_barrier()

            e_start(0, 0)
            e_wait(0, 0)
            adjust(0, gather_base)
            g_start(0)
            e_start(1, 1)

            def ebody(jj, _):
                j0 = NSLOT * jj
                for k in range(NSLOT):
                    j = j0 + k
                    o = (k + 1) % NSLOT
                    g_wait(k)
                    e_wait(o, j + 1)
                    adjust(o, gather_base)
                    g_start(o)
                    mul_rows(k)
                    scatter(k)
                    e_start(k, j + 2)
                return _
            lax.fori_loop(0, NCHUNK // NSLOT, ebody, 0)
            g_wait(0)
            e_wait(1, NCHUNK + 1)

            plsc.subcore_barrier()
            writeback(i + 1, out_base)
            plsc.subcore_barrier()
            return _
        lax.fori_loop(0, LAYERS, layer, 0)

        def mbody(p, _):
            off = r0 + p * PIECE
            hb = half_off + off
            pltpu.sync_copy(t_out.at[pl.ds(hb, PIECE)], pbuf)
            for li in (1, 2, 3):
                pltpu.sync_copy(t_out.at[pl.ds(li * SLAB + hb, PIECE)], p1)
                scale = 0.25 if li == 3 else 1.0

                def rbody(r, _, scale=scale):
                    for h in range(HALF // LANES):
                        sl = pl.ds(h * LANES, LANES)
                        pbuf[r, sl] = (pbuf[r, sl] + p1[r, sl]) * scale
                    return _
                lax.fori_loop(0, PIECE, rbody, 0)
            write_final(True, 0, off, pbuf)
            return _
        lax.fori_loop(0, NPIECE, mbody, 0)

    return lgcn_kernel


_LGCN = _build_kernel()


def kernel(user_emb, item_emb, adj_values, adj_indices):
    emb0 = jnp.concatenate([user_emb, item_emb], axis=0)
    rpad = NP - NODES
    table0 = jnp.concatenate(
        [jnp.pad(emb0[:, :HALF], ((0, rpad), (0, 0))),
         jnp.pad(emb0[:, HALF:], ((0, rpad), (0, 0)))], axis=0)

    row = adj_indices[0]
    col = adj_indices[1]
    pad = NS * NCHUNK * CHUNK - EDGES
    colp = jnp.pad(col, (0, pad))
    rowp = jnp.pad(row, (0, pad))
    valp = jnp.pad(adj_values, (0, pad))
    blob = jnp.stack([colp, rowp], axis=0)
    blob = blob.reshape(2, NS, NCHUNK, CHUNK).transpose(1, 2, 0, 3)
    blob = jnp.pad(blob, ((0, 0), (0, NCHUNK_ALLOC - NCHUNK), (0, 0), (0, 0)))
    vals = valp.reshape(NS, NCHUNK, 1, CHUNK)
    vals = jnp.pad(vals, ((0, 0), (0, NCHUNK_ALLOC - NCHUNK), (0, 0), (0, 0)))

    _, all_emb, final = _LGCN(table0, blob, vals)
    return (final[:USER_N], final[USER_N:], all_emb)
